# baseline (speedup 1.0000x reference)
import jax, jax.numpy as jnp
from jax.experimental import pallas as pl
from jax.experimental.pallas import tpu as pltpu

_B = 2000
_NS = 5


def _gcn_kernel(x_ref, w_ref, b_ref, o_hbm, o_v, sems):
    i = pl.program_id(0)
    buf = jax.lax.rem(i, 2)

    @pl.when(i >= 2)
    def _wait_prev():
        pltpu.make_async_copy(
            o_v.at[buf], o_hbm.at[pl.ds((i - 2) * _B, _B), :], sems.at[buf]
        ).wait()

    o_v[buf] = (
        jnp.dot(x_ref[...], w_ref[...], preferred_element_type=jnp.float32)
        + b_ref[...]
    )
    pltpu.make_async_copy(
        o_v.at[buf], o_hbm.at[pl.ds(i * _B, _B), :], sems.at[buf]
    ).start()

    @pl.when(i == _NS - 1)
    def _drain():
        pltpu.make_async_copy(
            o_v.at[1 - buf], o_hbm.at[pl.ds((i - 1) * _B, _B), :], sems.at[1 - buf]
        ).wait()
        pltpu.make_async_copy(
            o_v.at[buf], o_hbm.at[pl.ds(i * _B, _B), :], sems.at[buf]
        ).wait()


def kernel(An, X, weight, bias):
    del An
    n, d = X.shape
    units = weight.shape[1]
    bias2d = bias.reshape(1, units)
    return pl.pallas_call(
        _gcn_kernel,
        grid=(_NS,),
        in_specs=[
            pl.BlockSpec((_B, d), lambda i: (i, 0)),
            pl.BlockSpec((d, units), lambda i: (0, 0)),
            pl.BlockSpec((1, units), lambda i: (0, 0)),
        ],
        out_specs=pl.BlockSpec(memory_space=pltpu.MemorySpace.HBM),
        out_shape=jax.ShapeDtypeStruct((n, units), jnp.float32),
        scratch_shapes=[
            pltpu.MemorySpace.VMEM((2, _B, units), jnp.float32),
            pltpu.SemaphoreType.DMA((2,)),
        ],
    )(X, weight, bias2d)


# speedup vs baseline: 1.2143x; 1.2143x over previous
import jax, jax.numpy as jnp
from jax.experimental import pallas as pl
from jax.experimental.pallas import tpu as pltpu

_IN_CHUNKS = ((0, 1000), (1000, 4000), (5000, 5000))
_OUT_CHUNKS = ((0, 1000, 0), (1000, 4000, 1), (5000, 4000, 2), (9000, 1000, 2))


def _gcn_kernel(x_hbm, w_ref, b_ref, o_hbm, x_v, o_v, in_sems, out_sems):
    for c, (base, size) in enumerate(_IN_CHUNKS):
        rows = pl.ds(base, size)
        pltpu.make_async_copy(
            x_hbm.at[rows, :], x_v.at[rows, :], in_sems.at[c]
        ).start()
    w = w_ref[...]
    b = b_ref[...]
    waited = set()
    for c, (base, size, dep) in enumerate(_OUT_CHUNKS):
        rows = pl.ds(base, size)
        if dep not in waited:
            waited.add(dep)
            ib, isz = _IN_CHUNKS[dep]
            irows = pl.ds(ib, isz)
            pltpu.make_async_copy(
                x_hbm.at[irows, :], x_v.at[irows, :], in_sems.at[dep]
            ).wait()
        o_v[rows, :] = (
            jnp.dot(x_v[rows, :], w, preferred_element_type=jnp.float32) + b
        )
        pltpu.make_async_copy(
            o_v.at[rows, :], o_hbm.at[rows, :], out_sems.at[c]
        ).start()
    for c, (base, size, _) in enumerate(_OUT_CHUNKS):
        rows = pl.ds(base, size)
        pltpu.make_async_copy(
            o_v.at[rows, :], o_hbm.at[rows, :], out_sems.at[c]
        ).wait()


def kernel(An, X, weight, bias):
    del An
    n, d = X.shape
    units = weight.shape[1]
    bias2d = bias.reshape(1, units)
    return pl.pallas_call(
        _gcn_kernel,
        in_specs=[
            pl.BlockSpec(memory_space=pltpu.MemorySpace.HBM),
            pl.BlockSpec(memory_space=pltpu.MemorySpace.VMEM),
            pl.BlockSpec(memory_space=pltpu.MemorySpace.VMEM),
        ],
        out_specs=pl.BlockSpec(memory_space=pltpu.MemorySpace.HBM),
        out_shape=jax.ShapeDtypeStruct((n, units), jnp.float32),
        scratch_shapes=[
            pltpu.MemorySpace.VMEM((n, d), jnp.float32),
            pltpu.MemorySpace.VMEM((n, units), jnp.float32),
            pltpu.SemaphoreType.DMA((len(_IN_CHUNKS),)),
            pltpu.SemaphoreType.DMA((len(_OUT_CHUNKS),)),
        ],
    )(X, weight, bias2d)


# speedup vs baseline: 1.3226x; 1.0892x over previous
import jax, jax.numpy as jnp
from jax.experimental import pallas as pl

_B = 8000

def _gcn_kernel(x_ref, w_ref, b_ref, o_ref):
    o_ref[...] = (
        jnp.dot(x_ref[...], w_ref[...], preferred_element_type=jnp.float32)
        + b_ref[...]
    )

def kernel(An, X, weight, bias):
    del An
    n, d = X.shape
    units = weight.shape[1]
    bias2d = bias.reshape(1, units)
    return pl.pallas_call(
        _gcn_kernel,
        grid=(pl.cdiv(n, _B),),
        in_specs=[
            pl.BlockSpec((_B, d), lambda i: (i, 0)),
            pl.BlockSpec((d, units), lambda i: (0, 0)),
            pl.BlockSpec((1, units), lambda i: (0, 0)),
        ],
        out_specs=pl.BlockSpec((_B, units), lambda i: (i, 0)),
        out_shape=jax.ShapeDtypeStruct((n, units), jnp.float32),
    )(X, weight, bias2d)
